# Initial kernel scaffold; baseline (speedup 1.0000x reference)
import jax
import jax.numpy as jnp
from jax.experimental import pallas as pl


def kernel(x, edge_index, W0, b0, W1, b1, W2, b2, Wg, a_src, a_dst, bg, C1, c1b, C2, c2b, R1, r1b, R2, r2b):
    raise NotImplementedError("write your pallas kernel here")


# speedup vs baseline: 32.5593x; 32.5593x over previous
import functools

import jax
import jax.numpy as jnp
from jax import lax
from jax.experimental import pallas as pl
from jax.experimental.pallas import tpu as pltpu
from jax.experimental.pallas import tpu_sc as plsc

NN = 10000
D_IN = 128
HID = 64
HEADS = 4

NC, NS, LANES = 2, 16, 16
NW = NC * NS
CH = 128
NPAD = 10112
IBLK = 32
RPT = NPAD // NS

_mesh = plsc.VectorSubcoreMesh(
    core_axis_name="c", subcore_axis_name="s", num_cores=NC, num_subcores=NS)

_f32 = jnp.float32
_i32 = jnp.int32


def _worker_ids():
  c = lax.axis_index("c")
  s = lax.axis_index("s")
  return c, s, c * NS + s


def _zero_vmem(buf, rows, cols):
  z = jnp.zeros((LANES,), _f32)

  def body(i, carry):
    for q in range(cols // LANES):
      buf[i, pl.ds(q * LANES, LANES)] = z
    return carry

  lax.fori_loop(0, rows, body, 0)


def _zero_shared_slice(sh, zbuf):
  s = lax.axis_index("s")
  base = s * RPT
  n_full = RPT // CH
  rem = RPT - n_full * CH
  for m in range(n_full):
    pltpu.sync_copy(zbuf, sh.at[pl.ds(base + m * CH, CH)])
  if rem:
    pltpu.sync_copy(zbuf.at[pl.ds(0, rem)], sh.at[pl.ds(base + n_full * CH, rem)])


def _writeout_shared(sh, out_hbm, c):
  s = lax.axis_index("s")
  base = s * RPT
  pltpu.sync_copy(sh.at[pl.ds(base, RPT)], out_hbm.at[c, pl.ds(base, RPT)])


def _deg_body(nchunks, dst_hbm, out_hbm, idx_d, ones_v, zbuf, deg_sh):
  c, s, w = _worker_ids()
  one = jnp.ones((LANES,), _f32)

  def initbody(i, carry):
    ones_v[i, pl.ds(0, LANES)] = one
    zbuf[i, pl.ds(0, LANES)] = one * 0.0
    return carry

  lax.fori_loop(0, CH, initbody, 0)
  pltpu.sync_copy(dst_hbm.at[w], idx_d)
  _zero_shared_slice(deg_sh, zbuf)
  plsc.subcore_barrier()

  def body(j, carry):
    pltpu.sync_copy(ones_v, deg_sh.at[idx_d.at[j]], add=True)
    return carry

  lax.fori_loop(0, nchunks, body, 0)
  plsc.subcore_barrier()
  _writeout_shared(deg_sh, out_hbm, c)


def _gcn_body(nchunks, src_hbm, dst_hbm, tab_hbm, out_hbm,
              idx_s, idx_d, gbuf, gsem, acc_sh):
  c, s, w = _worker_ids()
  _zero_vmem(gbuf, CH, HID)
  pltpu.sync_copy(src_hbm.at[w], idx_s)
  pltpu.sync_copy(dst_hbm.at[w], idx_d)
  _zero_shared_slice(acc_sh, gbuf)
  plsc.subcore_barrier()

  def body(j, carry):
    pltpu.async_copy(tab_hbm.at[idx_s.at[j]], gbuf, gsem).wait()
    pltpu.sync_copy(gbuf, acc_sh.at[idx_d.at[j]], add=True)
    return carry

  lax.fori_loop(0, nchunks, body, 0)
  plsc.subcore_barrier()
  _writeout_shared(acc_sh, out_hbm, c)


def _gata_body(nchunks, src_hbm, dst_hbm, tabA_hbm, tabS_hbm, tabD_hbm, mvec_hbm,
               acc_out, den_out, ee_out,
               idx_s, idx_d, gA, gS, gD, dbuf, mv, gsem, ssem, dsem,
               acc_sh, den_sh):
  c, s, w = _worker_ids()
  pltpu.sync_copy(mvec_hbm, mv)
  _zero_vmem(gA, CH, 2 * HID)
  _zero_shared_slice(acc_sh, gA)
  _zero_vmem(dbuf, CH, LANES)
  _zero_shared_slice(den_sh, dbuf)
  plsc.subcore_barrier()
  mvv = mv[0, pl.ds(0, LANES)]

  def chunk(j, jj):
    pltpu.async_copy(tabA_hbm.at[idx_s.at[jj]], gA, gsem).wait()
    pltpu.async_copy(tabS_hbm.at[idx_s.at[jj]], gS, ssem).wait()
    pltpu.async_copy(tabD_hbm.at[idx_d.at[jj]], gD, dsem).wait()
    for k in range(CH // LANES):
      rows = lax.iota(_i32, LANES) + (LANES * k)
      for h in range(HEADS):
        colS = jnp.full((LANES,), h, _i32)
        colD = jnp.full((LANES,), h, _i32)
        sv = plsc.load_gather(gS, [rows, colS])
        dv = plsc.load_gather(gD, [rows, colD])
        es = sv + dv
        e = jnp.where(es > 0.0, es, es * 0.2)
        plsc.store_scatter(dbuf, [rows, colD], jnp.exp(e - mvv))
    hsel = jnp.bitwise_and(lax.iota(_i32, LANES), 3)

    def sc_body(i, carry2):
      ev = plsc.load_gather(dbuf, [jnp.full((LANES,), i, _i32), hsel])
      for h in range(2):
        svec = jnp.full((LANES,), ev[h], _f32)
        for q in range(HID // LANES):
          slc = pl.ds(h * HID + q * LANES, LANES)
          gA[i, slc] = gA[i, slc] * svec
      return carry2

    lax.fori_loop(0, CH, sc_body, 0)
    pltpu.sync_copy(gA, acc_sh.at[idx_d.at[jj]], add=True)
    pltpu.sync_copy(dbuf, den_sh.at[idx_d.at[jj]], add=True)
    pltpu.sync_copy(dbuf, ee_out.at[w, j])

  nfull = nchunks // IBLK
  rem = nchunks - nfull * IBLK

  def outer(b, carry):
    pltpu.sync_copy(src_hbm.at[w, pl.ds(b * IBLK, IBLK)], idx_s)
    pltpu.sync_copy(dst_hbm.at[w, pl.ds(b * IBLK, IBLK)], idx_d)

    def inner(jj, c2):
      chunk(b * IBLK + jj, jj)
      return c2

    lax.fori_loop(0, IBLK, inner, 0)
    return carry

  lax.fori_loop(0, nfull, outer, 0)
  if rem:
    pltpu.sync_copy(src_hbm.at[w, pl.ds(nfull * IBLK, rem)],
                    idx_s.at[pl.ds(0, rem)])
    pltpu.sync_copy(dst_hbm.at[w, pl.ds(nfull * IBLK, rem)],
                    idx_d.at[pl.ds(0, rem)])

    def inner_rem(jj, c2):
      chunk(nfull * IBLK + jj, jj)
      return c2

    lax.fori_loop(0, rem, inner_rem, 0)
  plsc.subcore_barrier()
  _writeout_shared(acc_sh, acc_out, c)
  _writeout_shared(den_sh, den_out, c)


def _gatb_body(nchunks, src_hbm, dst_hbm, tabB_hbm, ee_hbm, acc_out,
               idx_s, idx_d, gB, dbuf, gsem, acc_sh):
  c, s, w = _worker_ids()
  pltpu.sync_copy(src_hbm.at[w], idx_s)
  pltpu.sync_copy(dst_hbm.at[w], idx_d)
  _zero_vmem(gB, CH, 2 * HID)
  _zero_shared_slice(acc_sh, gB)
  plsc.subcore_barrier()

  def body(j, carry):
    pltpu.async_copy(tabB_hbm.at[idx_s.at[j]], gB, gsem).wait()
    pltpu.sync_copy(ee_hbm.at[w, j], dbuf)

    hsel = jnp.bitwise_and(lax.iota(_i32, LANES), 3)

    def sc_body(i, carry2):
      ev = plsc.load_gather(dbuf, [jnp.full((LANES,), i, _i32), hsel])
      for h in range(2):
        svec = jnp.full((LANES,), ev[2 + h], _f32)
        for q in range(HID // LANES):
          slc = pl.ds(h * HID + q * LANES, LANES)
          gB[i, slc] = gB[i, slc] * svec
      return carry2

    lax.fori_loop(0, CH, sc_body, 0)
    pltpu.sync_copy(gB, acc_sh.at[idx_d.at[j]], add=True)
    return carry

  lax.fori_loop(0, nchunks, body, 0)
  plsc.subcore_barrier()
  _writeout_shared(acc_sh, acc_out, c)


def _tc_k1(x_ref, w0_ref, degp_ref, hws_ref, dinv_ref):
  deg = degp_ref[0, :NN, 0:1] + degp_ref[1, :NN, 0:1]
  dinv = jnp.where(deg > 0.0, lax.rsqrt(deg), 0.0)
  hw = jnp.dot(x_ref[...], w0_ref[...], preferred_element_type=_f32)
  hws_ref[...] = hw * dinv
  dinv_ref[...] = dinv


def _tc_gcn_post(residual, accp_ref, dinv_ref, b_ref, wn_ref, hprev_ref,
                 h_ref, hwsn_ref):
  acc = accp_ref[0, :NN, :] + accp_ref[1, :NN, :]
  dinv = dinv_ref[...]
  h = jnp.maximum(acc * dinv + b_ref[...], 0.0)
  if residual:
    h = h + hprev_ref[...]
  h_ref[...] = h
  hwsn_ref[...] = jnp.dot(h, wn_ref[...], preferred_element_type=_f32) * dinv


def _tc_k7(accp_ref, dinv_ref, b2_ref, hprev_ref, wg_ref, as_ref, ad_ref,
           tabA_ref, tabB_ref, tabS_ref, tabD_ref, mvec_ref):
  acc = accp_ref[0, :NN, :] + accp_ref[1, :NN, :]
  h3 = hprev_ref[...] + jnp.maximum(acc * dinv_ref[...] + b2_ref[...], 0.0)
  hp = jnp.dot(h3, wg_ref[...], preferred_element_type=_f32)
  al_s = jnp.dot(hp, as_ref[...], preferred_element_type=_f32)
  al_d = jnp.dot(hp, ad_ref[...], preferred_element_type=_f32)
  m = jnp.maximum(jnp.max(al_s) + jnp.max(al_d), 0.0)
  tabA_ref[...] = hp[:, 0:128]
  tabB_ref[...] = hp[:, 128:256]
  tabS_ref[...] = jnp.zeros((NPAD, LANES), _f32)
  tabS_ref[0:NN, 0:4] = al_s
  tabD_ref[...] = jnp.zeros((NPAD, LANES), _f32)
  tabD_ref[0:NN, 0:4] = al_d
  mvec_ref[...] = jnp.full((1, LANES), m, _f32)


def _tc_k10(accA_ref, accB_ref, den_ref, bg_ref, c1_ref, c1b_ref, c2_ref,
            c2b_ref, r1_ref, r1b_ref, r2_ref, r2b_ref,
            cls_ref, rec_ref, h_ref):
  num0 = accA_ref[0, :NN, :] + accA_ref[1, :NN, :]
  num1 = accB_ref[0, :NN, :] + accB_ref[1, :NN, :]
  den = den_ref[0, :NN, 0:4] + den_ref[1, :NN, 0:4]
  h0 = num0[:, 0:HID] / (den[:, 0:1] + 1e-16)
  h1 = num0[:, HID:2 * HID] / (den[:, 1:2] + 1e-16)
  h2 = num1[:, 0:HID] / (den[:, 2:3] + 1e-16)
  h3 = num1[:, HID:2 * HID] / (den[:, 3:4] + 1e-16)
  h = 0.25 * (h0 + h1 + h2 + h3) + bg_ref[...]
  h_ref[...] = h
  hg = jnp.mean(h, axis=0, keepdims=True)
  hc = jnp.maximum(jnp.dot(hg, c1_ref[...], preferred_element_type=_f32)
                   + c1b_ref[...], 0.0)
  cls_ref[...] = jnp.dot(hc, c2_ref[...], preferred_element_type=_f32) + c2b_ref[...]
  hr = jnp.maximum(jnp.dot(h, r1_ref[...], preferred_element_type=_f32)
                   + r1b_ref[...], 0.0)
  rec_ref[...] = jnp.dot(hr, r2_ref[...], preferred_element_type=_f32) + r2b_ref[...]


def kernel(x, edge_index, W0, b0, W1, b1, W2, b2, Wg, a_src, a_dst, bg,
           C1, c1b, C2, c2b, R1, r1b, R2, r2b):
  n_edges = edge_index.shape[1]
  ep_tot = n_edges + NN
  nchunks = -(-ep_tot // (NW * CH))
  ep = NW * CH * nchunks
  npad_e = ep - ep_tot

  loops = jnp.arange(NN, dtype=_i32)
  src = jnp.concatenate([edge_index[0].astype(_i32), loops,
                         jnp.zeros((npad_e,), _i32)])
  dst = jnp.concatenate([edge_index[1].astype(_i32), loops,
                         jnp.full((npad_e,), NN, _i32)])
  srcw = src.reshape(NW, nchunks, CH)
  dstw = dst.reshape(NW, nchunks, CH)

  hmask = (jnp.arange(HEADS * HID)[:, None] // HID
           == jnp.arange(HEADS)[None, :]).astype(_f32)
  As = hmask * a_src.reshape(-1)[:, None]
  Ad = hmask * a_dst.reshape(-1)[:, None]

  deg_call = pl.kernel(
      functools.partial(_deg_body, nchunks),
      out_type=jax.ShapeDtypeStruct((NC, NPAD, LANES), _f32),
      mesh=_mesh,
      scratch_types=[
          pltpu.VMEM((nchunks, CH), _i32),
          pltpu.VMEM((CH, LANES), _f32),
          pltpu.VMEM((CH, LANES), _f32),
          pltpu.VMEM_SHARED((NPAD, LANES), _f32),
      ],
      compiler_params=pltpu.CompilerParams(use_tc_tiling_on_sc=False, needs_layout_passes=False),
      name="sc_degrees",
  )
  degp = deg_call(dstw)

  hws0, dinv = pl.pallas_call(
      _tc_k1,
      out_shape=[jax.ShapeDtypeStruct((NN, HID), _f32),
                 jax.ShapeDtypeStruct((NN, 1), _f32)],
      compiler_params=pltpu.CompilerParams(vmem_limit_bytes=120 * 2**20),
      name="tc_dinv_proj0",
  )(x, W0, degp)

  gcn_call = pl.kernel(
      functools.partial(_gcn_body, nchunks),
      out_type=jax.ShapeDtypeStruct((NC, NPAD, HID), _f32),
      mesh=_mesh,
      scratch_types=[
          pltpu.VMEM((nchunks, CH), _i32),
          pltpu.VMEM((nchunks, CH), _i32),
          pltpu.VMEM((CH, HID), _f32),
          pltpu.SemaphoreType.DMA,
          pltpu.VMEM_SHARED((NPAD, HID), _f32),
      ],
      compiler_params=pltpu.CompilerParams(use_tc_tiling_on_sc=False, needs_layout_passes=False),
      name="sc_gcn_edges",
  )

  def gcn_post(residual, accp, b, wn, hprev):
    return pl.pallas_call(
        functools.partial(_tc_gcn_post, residual),
        out_shape=[jax.ShapeDtypeStruct((NN, HID), _f32),
                   jax.ShapeDtypeStruct((NN, HID), _f32)],
        compiler_params=pltpu.CompilerParams(vmem_limit_bytes=120 * 2**20),
      name="tc_gcn_post",
    )(accp, dinv, b.reshape(1, HID), wn, hprev)

  accp0 = gcn_call(srcw, dstw, hws0)
  h1, hws1 = gcn_post(False, accp0, b0, W1, hws0)
  accp1 = gcn_call(srcw, dstw, hws1)
  h2, hws2 = gcn_post(True, accp1, b1, W2, h1)
  accp2 = gcn_call(srcw, dstw, hws2)

  tabA, tabB, tabS, tabD, mvec = pl.pallas_call(
      _tc_k7,
      out_shape=[jax.ShapeDtypeStruct((NN, 2 * HID), _f32),
                 jax.ShapeDtypeStruct((NN, 2 * HID), _f32),
                 jax.ShapeDtypeStruct((NPAD, LANES), _f32),
                 jax.ShapeDtypeStruct((NPAD, LANES), _f32),
                 jax.ShapeDtypeStruct((1, LANES), _f32)],
      compiler_params=pltpu.CompilerParams(vmem_limit_bytes=120 * 2**20),
      name="tc_gat_prep",
  )(accp2, dinv, b2.reshape(1, HID), h2, Wg, As, Ad)

  gata_call = pl.kernel(
      functools.partial(_gata_body, nchunks),
      out_type=[jax.ShapeDtypeStruct((NC, NPAD, 2 * HID), _f32),
                jax.ShapeDtypeStruct((NC, NPAD, LANES), _f32),
                jax.ShapeDtypeStruct((NW, nchunks, CH, LANES), _f32)],
      mesh=_mesh,
      scratch_types=[
          pltpu.VMEM((IBLK, CH), _i32),
          pltpu.VMEM((IBLK, CH), _i32),
          pltpu.VMEM((CH, 2 * HID), _f32),
          pltpu.VMEM((CH, LANES), _f32),
          pltpu.VMEM((CH, LANES), _f32),
          pltpu.VMEM((CH, LANES), _f32),
          pltpu.VMEM((1, LANES), _f32),
          pltpu.SemaphoreType.DMA,
          pltpu.SemaphoreType.DMA,
          pltpu.SemaphoreType.DMA,
          pltpu.VMEM_SHARED((NPAD, 2 * HID), _f32),
          pltpu.VMEM_SHARED((NPAD, LANES), _f32),
      ],
      compiler_params=pltpu.CompilerParams(use_tc_tiling_on_sc=False, needs_layout_passes=False),
      name="sc_gat_a",
  )
  accA, denp, ee = gata_call(srcw, dstw, tabA, tabS, tabD, mvec)

  gatb_call = pl.kernel(
      functools.partial(_gatb_body, nchunks),
      out_type=jax.ShapeDtypeStruct((NC, NPAD, 2 * HID), _f32),
      mesh=_mesh,
      scratch_types=[
          pltpu.VMEM((nchunks, CH), _i32),
          pltpu.VMEM((nchunks, CH), _i32),
          pltpu.VMEM((CH, 2 * HID), _f32),
          pltpu.VMEM((CH, LANES), _f32),
          pltpu.SemaphoreType.DMA,
          pltpu.VMEM_SHARED((NPAD, 2 * HID), _f32),
      ],
      compiler_params=pltpu.CompilerParams(use_tc_tiling_on_sc=False, needs_layout_passes=False),
      name="sc_gat_b",
  )
  accB = gatb_call(srcw, dstw, tabB, ee)

  cls, rec, h = pl.pallas_call(
      _tc_k10,
      out_shape=[jax.ShapeDtypeStruct((1, 2), _f32),
                 jax.ShapeDtypeStruct((NN, D_IN), _f32),
                 jax.ShapeDtypeStruct((NN, HID), _f32)],
      compiler_params=pltpu.CompilerParams(vmem_limit_bytes=120 * 2**20),
      name="tc_final",
  )(accA, accB, denp, bg.reshape(1, HID), C1, c1b.reshape(1, HID // 2),
    C2, c2b.reshape(1, 2), R1, r1b.reshape(1, HID), R2, r2b.reshape(1, D_IN))
  return (cls, rec, h)


# speedup vs baseline: 38.4332x; 1.1804x over previous
import functools

import jax
import jax.numpy as jnp
from jax import lax
from jax.experimental import pallas as pl
from jax.experimental.pallas import tpu as pltpu
from jax.experimental.pallas import tpu_sc as plsc

NN = 10000
D_IN = 128
HID = 64
HEADS = 4

NC, NS, LANES = 2, 16, 16
NW = NC * NS
CH = 128
NPAD = 10112
IBLK = 32
RPT = NPAD // NS

_mesh = plsc.VectorSubcoreMesh(
    core_axis_name="c", subcore_axis_name="s", num_cores=NC, num_subcores=NS)

_f32 = jnp.float32
_i32 = jnp.int32


def _worker_ids():
  c = lax.axis_index("c")
  s = lax.axis_index("s")
  return c, s, c * NS + s


def _zero_vmem(buf, rows, cols):
  z = jnp.zeros((LANES,), _f32)

  def body(i, carry):
    for q in range(cols // LANES):
      buf[i, pl.ds(q * LANES, LANES)] = z
    return carry

  lax.fori_loop(0, rows, body, 0)


def _zero_shared_slice(sh, zbuf):
  s = lax.axis_index("s")
  base = s * RPT
  n_full = RPT // CH
  rem = RPT - n_full * CH
  for m in range(n_full):
    pltpu.sync_copy(zbuf, sh.at[pl.ds(base + m * CH, CH)])
  if rem:
    pltpu.sync_copy(zbuf.at[pl.ds(0, rem)], sh.at[pl.ds(base + n_full * CH, rem)])


def _writeout_shared(sh, out_hbm, c):
  s = lax.axis_index("s")
  base = s * RPT
  pltpu.sync_copy(sh.at[pl.ds(base, RPT)], out_hbm.at[c, pl.ds(base, RPT)])


def _deg_body(nchunks, dst_hbm, out_hbm, idx_d, ones_v, zbuf, deg_sh):
  c, s, w = _worker_ids()
  one = jnp.ones((LANES,), _f32)

  def initbody(i, carry):
    ones_v[i, pl.ds(0, LANES)] = one
    zbuf[i, pl.ds(0, LANES)] = one * 0.0
    return carry

  lax.fori_loop(0, CH, initbody, 0)
  pltpu.sync_copy(dst_hbm.at[w], idx_d)
  _zero_shared_slice(deg_sh, zbuf)
  plsc.subcore_barrier()

  def body(j, carry):
    pltpu.sync_copy(ones_v, deg_sh.at[idx_d.at[j]], add=True)
    return carry

  lax.fori_loop(0, nchunks, body, 0)
  plsc.subcore_barrier()
  _writeout_shared(deg_sh, out_hbm, c)


def _gcn_body(nchunks, src_hbm, dst_hbm, tab_hbm, out_hbm,
              idx_s, idx_d, gbuf, gbuf2, gsem, gsem2, acc_sh):
  c, s, w = _worker_ids()
  _zero_vmem(gbuf, CH, HID)
  pltpu.sync_copy(src_hbm.at[w], idx_s)
  pltpu.sync_copy(dst_hbm.at[w], idx_d)
  _zero_shared_slice(acc_sh, gbuf)
  plsc.subcore_barrier()

  npairs = nchunks // 2

  def body(p, carry):
    j = p * 2
    cp0 = pltpu.async_copy(tab_hbm.at[idx_s.at[j]], gbuf, gsem)
    cp1 = pltpu.async_copy(tab_hbm.at[idx_s.at[j + 1]], gbuf2, gsem2)
    cp0.wait()
    pltpu.sync_copy(gbuf, acc_sh.at[idx_d.at[j]], add=True)
    cp1.wait()
    pltpu.sync_copy(gbuf2, acc_sh.at[idx_d.at[j + 1]], add=True)
    return carry

  lax.fori_loop(0, npairs, body, 0)
  if nchunks % 2:
    j = nchunks - 1
    pltpu.async_copy(tab_hbm.at[idx_s.at[j]], gbuf, gsem).wait()
    pltpu.sync_copy(gbuf, acc_sh.at[idx_d.at[j]], add=True)
  plsc.subcore_barrier()
  _writeout_shared(acc_sh, out_hbm, c)


def _gata_body(nchunks, src_hbm, dst_hbm, tabA_hbm, tabS_hbm, tabD_hbm, mvec_hbm,
               acc_out, den_out, ee_out,
               idx_s, idx_d, gA, gS, gD, dbuf, mv, gsem, ssem, dsem,
               acc_sh, den_sh):
  c, s, w = _worker_ids()
  pltpu.sync_copy(mvec_hbm, mv)
  _zero_vmem(gA, CH, 2 * HID)
  _zero_shared_slice(acc_sh, gA)
  _zero_vmem(dbuf, CH, LANES)
  _zero_shared_slice(den_sh, dbuf)
  plsc.subcore_barrier()
  mvv = mv[0, pl.ds(0, LANES)]

  def chunk(j, jj):
    cpA = pltpu.async_copy(tabA_hbm.at[idx_s.at[jj]], gA, gsem)
    cpS = pltpu.async_copy(tabS_hbm.at[idx_s.at[jj]], gS, ssem)
    cpD = pltpu.async_copy(tabD_hbm.at[idx_d.at[jj]], gD, dsem)
    cpS.wait()
    cpD.wait()
    for k in range(CH // LANES):
      rows = lax.iota(_i32, LANES) + (LANES * k)
      for h in range(HEADS):
        colS = jnp.full((LANES,), h, _i32)
        colD = jnp.full((LANES,), h, _i32)
        sv = plsc.load_gather(gS, [rows, colS])
        dv = plsc.load_gather(gD, [rows, colD])
        es = sv + dv
        e = jnp.where(es > 0.0, es, es * 0.2)
        plsc.store_scatter(dbuf, [rows, colD], jnp.exp(e - mvv))
    cpA.wait()
    hsel = jnp.bitwise_and(lax.iota(_i32, LANES), 3)

    def sc_body(i, carry2):
      ev = plsc.load_gather(dbuf, [jnp.full((LANES,), i, _i32), hsel])
      for h in range(2):
        svec = jnp.full((LANES,), ev[h], _f32)
        for q in range(HID // LANES):
          slc = pl.ds(h * HID + q * LANES, LANES)
          gA[i, slc] = gA[i, slc] * svec
      return carry2

    lax.fori_loop(0, CH, sc_body, 0)
    pltpu.sync_copy(gA, acc_sh.at[idx_d.at[jj]], add=True)
    pltpu.sync_copy(dbuf, den_sh.at[idx_d.at[jj]], add=True)
    pltpu.sync_copy(dbuf, ee_out.at[w, j])

  nfull = nchunks // IBLK
  rem = nchunks - nfull * IBLK

  def outer(b, carry):
    pltpu.sync_copy(src_hbm.at[w, pl.ds(b * IBLK, IBLK)], idx_s)
    pltpu.sync_copy(dst_hbm.at[w, pl.ds(b * IBLK, IBLK)], idx_d)

    def inner(jj, c2):
      chunk(b * IBLK + jj, jj)
      return c2

    lax.fori_loop(0, IBLK, inner, 0)
    return carry

  lax.fori_loop(0, nfull, outer, 0)
  if rem:
    pltpu.sync_copy(src_hbm.at[w, pl.ds(nfull * IBLK, rem)],
                    idx_s.at[pl.ds(0, rem)])
    pltpu.sync_copy(dst_hbm.at[w, pl.ds(nfull * IBLK, rem)],
                    idx_d.at[pl.ds(0, rem)])

    def inner_rem(jj, c2):
      chunk(nfull * IBLK + jj, jj)
      return c2

    lax.fori_loop(0, rem, inner_rem, 0)
  plsc.subcore_barrier()
  _writeout_shared(acc_sh, acc_out, c)
  _writeout_shared(den_sh, den_out, c)


def _gatb_body(nchunks, src_hbm, dst_hbm, tabB_hbm, ee_hbm, acc_out,
               idx_s, idx_d, gB, dbuf, gsem, acc_sh):
  c, s, w = _worker_ids()
  pltpu.sync_copy(src_hbm.at[w], idx_s)
  pltpu.sync_copy(dst_hbm.at[w], idx_d)
  _zero_vmem(gB, CH, 2 * HID)
  _zero_shared_slice(acc_sh, gB)
  plsc.subcore_barrier()

  def body(j, carry):
    cpB = pltpu.async_copy(tabB_hbm.at[idx_s.at[j]], gB, gsem)
    pltpu.sync_copy(ee_hbm.at[w, j], dbuf)
    cpB.wait()

    hsel = jnp.bitwise_and(lax.iota(_i32, LANES), 3)

    def sc_body(i, carry2):
      ev = plsc.load_gather(dbuf, [jnp.full((LANES,), i, _i32), hsel])
      for h in range(2):
        svec = jnp.full((LANES,), ev[2 + h], _f32)
        for q in range(HID // LANES):
          slc = pl.ds(h * HID + q * LANES, LANES)
          gB[i, slc] = gB[i, slc] * svec
      return carry2

    lax.fori_loop(0, CH, sc_body, 0)
    pltpu.sync_copy(gB, acc_sh.at[idx_d.at[j]], add=True)
    return carry

  lax.fori_loop(0, nchunks, body, 0)
  plsc.subcore_barrier()
  _writeout_shared(acc_sh, acc_out, c)


def _tc_k1(x_ref, w0_ref, degp_ref, hws_ref, dinv_ref):
  deg = degp_ref[0, :NN, 0:1] + degp_ref[1, :NN, 0:1]
  dinv = jnp.where(deg > 0.0, lax.rsqrt(deg), 0.0)
  hw = jnp.dot(x_ref[...], w0_ref[...], preferred_element_type=_f32)
  hws_ref[...] = hw * dinv
  dinv_ref[...] = dinv


def _tc_gcn_post(residual, accp_ref, dinv_ref, b_ref, wn_ref, hprev_ref,
                 h_ref, hwsn_ref):
  acc = accp_ref[0, :NN, :] + accp_ref[1, :NN, :]
  dinv = dinv_ref[...]
  h = jnp.maximum(acc * dinv + b_ref[...], 0.0)
  if residual:
    h = h + hprev_ref[...]
  h_ref[...] = h
  hwsn_ref[...] = jnp.dot(h, wn_ref[...], preferred_element_type=_f32) * dinv


def _tc_k7(accp_ref, dinv_ref, b2_ref, hprev_ref, wg_ref, as_ref, ad_ref,
           tabA_ref, tabB_ref, tabS_ref, tabD_ref, mvec_ref):
  acc = accp_ref[0, :NN, :] + accp_ref[1, :NN, :]
  h3 = hprev_ref[...] + jnp.maximum(acc * dinv_ref[...] + b2_ref[...], 0.0)
  hp = jnp.dot(h3, wg_ref[...], preferred_element_type=_f32)
  al_s = jnp.dot(hp, as_ref[...], preferred_element_type=_f32)
  al_d = jnp.dot(hp, ad_ref[...], preferred_element_type=_f32)
  m = jnp.maximum(jnp.max(al_s) + jnp.max(al_d), 0.0)
  tabA_ref[...] = hp[:, 0:128]
  tabB_ref[...] = hp[:, 128:256]
  tabS_ref[...] = jnp.zeros((NPAD, LANES), _f32)
  tabS_ref[0:NN, 0:4] = al_s
  tabD_ref[...] = jnp.zeros((NPAD, LANES), _f32)
  tabD_ref[0:NN, 0:4] = al_d
  mvec_ref[...] = jnp.full((1, LANES), m, _f32)


def _tc_k10(accA_ref, accB_ref, den_ref, bg_ref, c1_ref, c1b_ref, c2_ref,
            c2b_ref, r1_ref, r1b_ref, r2_ref, r2b_ref,
            cls_ref, rec_ref, h_ref):
  num0 = accA_ref[0, :NN, :] + accA_ref[1, :NN, :]
  num1 = accB_ref[0, :NN, :] + accB_ref[1, :NN, :]
  den = den_ref[0, :NN, 0:4] + den_ref[1, :NN, 0:4]
  h0 = num0[:, 0:HID] / (den[:, 0:1] + 1e-16)
  h1 = num0[:, HID:2 * HID] / (den[:, 1:2] + 1e-16)
  h2 = num1[:, 0:HID] / (den[:, 2:3] + 1e-16)
  h3 = num1[:, HID:2 * HID] / (den[:, 3:4] + 1e-16)
  h = 0.25 * (h0 + h1 + h2 + h3) + bg_ref[...]
  h_ref[...] = h
  hg = jnp.mean(h, axis=0, keepdims=True)
  hc = jnp.maximum(jnp.dot(hg, c1_ref[...], preferred_element_type=_f32)
                   + c1b_ref[...], 0.0)
  cls_ref[...] = jnp.dot(hc, c2_ref[...], preferred_element_type=_f32) + c2b_ref[...]
  hr = jnp.maximum(jnp.dot(h, r1_ref[...], preferred_element_type=_f32)
                   + r1b_ref[...], 0.0)
  rec_ref[...] = jnp.dot(hr, r2_ref[...], preferred_element_type=_f32) + r2b_ref[...]


def kernel(x, edge_index, W0, b0, W1, b1, W2, b2, Wg, a_src, a_dst, bg,
           C1, c1b, C2, c2b, R1, r1b, R2, r2b):
  n_edges = edge_index.shape[1]
  ep_tot = n_edges + NN
  nchunks = -(-ep_tot // (NW * CH))
  ep = NW * CH * nchunks
  npad_e = ep - ep_tot

  loops = jnp.arange(NN, dtype=_i32)
  src = jnp.concatenate([edge_index[0].astype(_i32), loops,
                         jnp.zeros((npad_e,), _i32)])
  dst = jnp.concatenate([edge_index[1].astype(_i32), loops,
                         jnp.full((npad_e,), NN, _i32)])
  srcw = src.reshape(NW, nchunks, CH)
  dstw = dst.reshape(NW, nchunks, CH)

  hmask = (jnp.arange(HEADS * HID)[:, None] // HID
           == jnp.arange(HEADS)[None, :]).astype(_f32)
  As = hmask * a_src.reshape(-1)[:, None]
  Ad = hmask * a_dst.reshape(-1)[:, None]

  deg_call = pl.kernel(
      functools.partial(_deg_body, nchunks),
      out_type=jax.ShapeDtypeStruct((NC, NPAD, LANES), _f32),
      mesh=_mesh,
      scratch_types=[
          pltpu.VMEM((nchunks, CH), _i32),
          pltpu.VMEM((CH, LANES), _f32),
          pltpu.VMEM((CH, LANES), _f32),
          pltpu.VMEM_SHARED((NPAD, LANES), _f32),
      ],
      compiler_params=pltpu.CompilerParams(use_tc_tiling_on_sc=False, needs_layout_passes=False),
      name="sc_degrees",
  )
  degp = deg_call(dstw)

  hws0, dinv = pl.pallas_call(
      _tc_k1,
      out_shape=[jax.ShapeDtypeStruct((NN, HID), _f32),
                 jax.ShapeDtypeStruct((NN, 1), _f32)],
      compiler_params=pltpu.CompilerParams(vmem_limit_bytes=120 * 2**20),
      name="tc_dinv_proj0",
  )(x, W0, degp)

  gcn_call = pl.kernel(
      functools.partial(_gcn_body, nchunks),
      out_type=jax.ShapeDtypeStruct((NC, NPAD, HID), _f32),
      mesh=_mesh,
      scratch_types=[
          pltpu.VMEM((nchunks, CH), _i32),
          pltpu.VMEM((nchunks, CH), _i32),
          pltpu.VMEM((CH, HID), _f32),
          pltpu.VMEM((CH, HID), _f32),
          pltpu.SemaphoreType.DMA,
          pltpu.SemaphoreType.DMA,
          pltpu.VMEM_SHARED((NPAD, HID), _f32),
      ],
      compiler_params=pltpu.CompilerParams(use_tc_tiling_on_sc=False, needs_layout_passes=False),
      name="sc_gcn_edges",
  )

  def gcn_post(residual, accp, b, wn, hprev):
    return pl.pallas_call(
        functools.partial(_tc_gcn_post, residual),
        out_shape=[jax.ShapeDtypeStruct((NN, HID), _f32),
                   jax.ShapeDtypeStruct((NN, HID), _f32)],
        compiler_params=pltpu.CompilerParams(vmem_limit_bytes=120 * 2**20),
      name="tc_gcn_post",
    )(accp, dinv, b.reshape(1, HID), wn, hprev)

  accp0 = gcn_call(srcw, dstw, hws0)
  h1, hws1 = gcn_post(False, accp0, b0, W1, hws0)
  accp1 = gcn_call(srcw, dstw, hws1)
  h2, hws2 = gcn_post(True, accp1, b1, W2, h1)
  accp2 = gcn_call(srcw, dstw, hws2)

  tabA, tabB, tabS, tabD, mvec = pl.pallas_call(
      _tc_k7,
      out_shape=[jax.ShapeDtypeStruct((NN, 2 * HID), _f32),
                 jax.ShapeDtypeStruct((NN, 2 * HID), _f32),
                 jax.ShapeDtypeStruct((NPAD, LANES), _f32),
                 jax.ShapeDtypeStruct((NPAD, LANES), _f32),
                 jax.ShapeDtypeStruct((1, LANES), _f32)],
      compiler_params=pltpu.CompilerParams(vmem_limit_bytes=120 * 2**20),
      name="tc_gat_prep",
  )(accp2, dinv, b2.reshape(1, HID), h2, Wg, As, Ad)

  gata_call = pl.kernel(
      functools.partial(_gata_body, nchunks),
      out_type=[jax.ShapeDtypeStruct((NC, NPAD, 2 * HID), _f32),
                jax.ShapeDtypeStruct((NC, NPAD, LANES), _f32),
                jax.ShapeDtypeStruct((NW, nchunks, CH, LANES), _f32)],
      mesh=_mesh,
      scratch_types=[
          pltpu.VMEM((IBLK, CH), _i32),
          pltpu.VMEM((IBLK, CH), _i32),
          pltpu.VMEM((CH, 2 * HID), _f32),
          pltpu.VMEM((CH, LANES), _f32),
          pltpu.VMEM((CH, LANES), _f32),
          pltpu.VMEM((CH, LANES), _f32),
          pltpu.VMEM((1, LANES), _f32),
          pltpu.SemaphoreType.DMA,
          pltpu.SemaphoreType.DMA,
          pltpu.SemaphoreType.DMA,
          pltpu.VMEM_SHARED((NPAD, 2 * HID), _f32),
          pltpu.VMEM_SHARED((NPAD, LANES), _f32),
      ],
      compiler_params=pltpu.CompilerParams(use_tc_tiling_on_sc=False, needs_layout_passes=False),
      name="sc_gat_a",
  )
  accA, denp, ee = gata_call(srcw, dstw, tabA, tabS, tabD, mvec)

  gatb_call = pl.kernel(
      functools.partial(_gatb_body, nchunks),
      out_type=jax.ShapeDtypeStruct((NC, NPAD, 2 * HID), _f32),
      mesh=_mesh,
      scratch_types=[
          pltpu.VMEM((nchunks, CH), _i32),
          pltpu.VMEM((nchunks, CH), _i32),
          pltpu.VMEM((CH, 2 * HID), _f32),
          pltpu.VMEM((CH, LANES), _f32),
          pltpu.SemaphoreType.DMA,
          pltpu.VMEM_SHARED((NPAD, 2 * HID), _f32),
      ],
      compiler_params=pltpu.CompilerParams(use_tc_tiling_on_sc=False, needs_layout_passes=False),
      name="sc_gat_b",
  )
  accB = gatb_call(srcw, dstw, tabB, ee)

  cls, rec, h = pl.pallas_call(
      _tc_k10,
      out_shape=[jax.ShapeDtypeStruct((1, 2), _f32),
                 jax.ShapeDtypeStruct((NN, D_IN), _f32),
                 jax.ShapeDtypeStruct((NN, HID), _f32)],
      compiler_params=pltpu.CompilerParams(vmem_limit_bytes=120 * 2**20),
      name="tc_final",
  )(accA, accB, denp, bg.reshape(1, HID), C1, c1b.reshape(1, HID // 2),
    C2, c2b.reshape(1, 2), R1, r1b.reshape(1, HID), R2, r2b.reshape(1, D_IN))
  return (cls, rec, h)


# speedup vs baseline: 38.9829x; 1.0143x over previous
import functools

import jax
import jax.numpy as jnp
from jax import lax
from jax.experimental import pallas as pl
from jax.experimental.pallas import tpu as pltpu
from jax.experimental.pallas import tpu_sc as plsc

NN = 10000
D_IN = 128
HID = 64
HEADS = 4

NC, NS, LANES = 2, 16, 16
NW = NC * NS
CH = 128
NPAD = 10112
IBLK = 32
RPT = NPAD // NS

_mesh = plsc.VectorSubcoreMesh(
    core_axis_name="c", subcore_axis_name="s", num_cores=NC, num_subcores=NS)

_f32 = jnp.float32
_i32 = jnp.int32


def _worker_ids():
  c = lax.axis_index("c")
  s = lax.axis_index("s")
  return c, s, c * NS + s


def _zero_vmem(buf, rows, cols):
  z = jnp.zeros((LANES,), _f32)

  def body(i, carry):
    for q in range(cols // LANES):
      buf[i, pl.ds(q * LANES, LANES)] = z
    return carry

  lax.fori_loop(0, rows, body, 0)


def _zero_shared_slice(sh, zbuf):
  s = lax.axis_index("s")
  base = s * RPT
  n_full = RPT // CH
  rem = RPT - n_full * CH
  for m in range(n_full):
    pltpu.sync_copy(zbuf, sh.at[pl.ds(base + m * CH, CH)])
  if rem:
    pltpu.sync_copy(zbuf.at[pl.ds(0, rem)], sh.at[pl.ds(base + n_full * CH, rem)])


def _writeout_shared(sh, out_hbm, c):
  s = lax.axis_index("s")
  base = s * RPT
  pltpu.sync_copy(sh.at[pl.ds(base, RPT)], out_hbm.at[c, pl.ds(base, RPT)])


def _deg_body(nchunks, dst_hbm, out_hbm, idx_d, ones_v, zbuf, deg_sh):
  c, s, w = _worker_ids()
  one = jnp.ones((LANES,), _f32)

  def initbody(i, carry):
    ones_v[i, pl.ds(0, LANES)] = one
    zbuf[i, pl.ds(0, LANES)] = one * 0.0
    return carry

  lax.fori_loop(0, CH, initbody, 0)
  pltpu.sync_copy(dst_hbm.at[w], idx_d)
  _zero_shared_slice(deg_sh, zbuf)
  plsc.subcore_barrier()

  def body(j, carry):
    pltpu.sync_copy(ones_v, deg_sh.at[idx_d.at[j]], add=True)
    return carry

  lax.fori_loop(0, nchunks, body, 0)
  plsc.subcore_barrier()
  _writeout_shared(deg_sh, out_hbm, c)


def _gcn_body(nchunks, src_hbm, dst_hbm, tab_hbm, out_hbm,
              idx_s, idx_d, gbuf, gbuf2, gsem, gsem2, acc_sh):
  c, s, w = _worker_ids()
  _zero_vmem(gbuf, CH, HID)
  pltpu.sync_copy(src_hbm.at[w], idx_s)
  pltpu.sync_copy(dst_hbm.at[w], idx_d)
  _zero_shared_slice(acc_sh, gbuf)
  plsc.subcore_barrier()

  npairs = nchunks // 2

  def body(p, carry):
    j = p * 2
    cp0 = pltpu.async_copy(tab_hbm.at[idx_s.at[j]], gbuf, gsem)
    cp1 = pltpu.async_copy(tab_hbm.at[idx_s.at[j + 1]], gbuf2, gsem2)
    cp0.wait()
    pltpu.sync_copy(gbuf, acc_sh.at[idx_d.at[j]], add=True)
    cp1.wait()
    pltpu.sync_copy(gbuf2, acc_sh.at[idx_d.at[j + 1]], add=True)
    return carry

  lax.fori_loop(0, npairs, body, 0)
  if nchunks % 2:
    j = nchunks - 1
    pltpu.async_copy(tab_hbm.at[idx_s.at[j]], gbuf, gsem).wait()
    pltpu.sync_copy(gbuf, acc_sh.at[idx_d.at[j]], add=True)
  plsc.subcore_barrier()
  _writeout_shared(acc_sh, out_hbm, c)


def _gata_body(nchunks, src_hbm, dst_hbm, tabA_hbm, tabS_hbm, tabD_hbm, mvec_hbm,
               acc_out, den_out, ee_out,
               idx_s, idx_d, gA, gS, gD, dbuf, mv, gsem, ssem, dsem,
               acc_sh, den_sh):
  c, s, w = _worker_ids()
  pltpu.sync_copy(mvec_hbm, mv)
  _zero_vmem(gA, CH, 2 * HID)
  _zero_shared_slice(acc_sh, gA)
  _zero_vmem(dbuf, CH, LANES)
  _zero_shared_slice(den_sh, dbuf)
  plsc.subcore_barrier()
  mvv = mv[0, pl.ds(0, LANES)]

  def chunk(j, jj):
    cpA = pltpu.async_copy(tabA_hbm.at[idx_s.at[jj]], gA, gsem)
    cpS = pltpu.async_copy(tabS_hbm.at[idx_s.at[jj]], gS, ssem)
    cpD = pltpu.async_copy(tabD_hbm.at[idx_d.at[jj]], gD, dsem)
    cpS.wait()
    cpD.wait()
    for k in range(CH // LANES):
      rows = lax.iota(_i32, LANES) + (LANES * k)
      for h in range(HEADS):
        colS = jnp.full((LANES,), h, _i32)
        colD = jnp.full((LANES,), h, _i32)
        sv = plsc.load_gather(gS, [rows, colS])
        dv = plsc.load_gather(gD, [rows, colD])
        es = sv + dv
        e = jnp.where(es > 0.0, es, es * 0.2)
        plsc.store_scatter(dbuf, [rows, colD], jnp.exp(e - mvv))
    cpA.wait()
    hsel = jnp.bitwise_and(lax.iota(_i32, LANES), 3)

    def sc_body(i, carry2):
      ev = plsc.load_gather(dbuf, [jnp.full((LANES,), i, _i32), hsel])
      for h in range(2):
        svec = jnp.full((LANES,), ev[h], _f32)
        for q in range(HID // LANES):
          slc = pl.ds(h * HID + q * LANES, LANES)
          gA[i, slc] = gA[i, slc] * svec
      return carry2

    lax.fori_loop(0, CH, sc_body, 0)
    pltpu.sync_copy(gA, acc_sh.at[idx_d.at[jj]], add=True)
    pltpu.sync_copy(dbuf, den_sh.at[idx_d.at[jj]], add=True)
    pltpu.sync_copy(dbuf, ee_out.at[w, j])

  nfull = nchunks // IBLK
  rem = nchunks - nfull * IBLK

  def outer(b, carry):
    pltpu.sync_copy(src_hbm.at[w, pl.ds(b * IBLK, IBLK)], idx_s)
    pltpu.sync_copy(dst_hbm.at[w, pl.ds(b * IBLK, IBLK)], idx_d)

    def inner(jj, c2):
      chunk(b * IBLK + jj, jj)
      return c2

    lax.fori_loop(0, IBLK, inner, 0)
    return carry

  lax.fori_loop(0, nfull, outer, 0)
  if rem:
    pltpu.sync_copy(src_hbm.at[w, pl.ds(nfull * IBLK, rem)],
                    idx_s.at[pl.ds(0, rem)])
    pltpu.sync_copy(dst_hbm.at[w, pl.ds(nfull * IBLK, rem)],
                    idx_d.at[pl.ds(0, rem)])

    def inner_rem(jj, c2):
      chunk(nfull * IBLK + jj, jj)
      return c2

    lax.fori_loop(0, rem, inner_rem, 0)
  plsc.subcore_barrier()
  _writeout_shared(acc_sh, acc_out, c)
  _writeout_shared(den_sh, den_out, c)


def _gatb_body(nchunks, src_hbm, dst_hbm, tabB_hbm, ee_hbm, acc_out,
               idx_s, idx_d, gB, gB2, dbuf, dbuf2, gsem, gsem2, acc_sh):
  c, s, w = _worker_ids()
  _zero_vmem(gB, CH, 2 * HID)
  _zero_shared_slice(acc_sh, gB)
  plsc.subcore_barrier()
  hsel = jnp.bitwise_and(lax.iota(_i32, LANES), 3)

  def scale(gX, dX):
    def sc_body(i, carry2):
      ev = plsc.load_gather(dX, [jnp.full((LANES,), i, _i32), hsel])
      for h in range(2):
        svec = jnp.full((LANES,), ev[2 + h], _f32)
        for q in range(HID // LANES):
          slc = pl.ds(h * HID + q * LANES, LANES)
          gX[i, slc] = gX[i, slc] * svec
      return carry2

    lax.fori_loop(0, CH, sc_body, 0)

  def pair(j, jj):
    cp0 = pltpu.async_copy(tabB_hbm.at[idx_s.at[jj]], gB, gsem)
    cp1 = pltpu.async_copy(tabB_hbm.at[idx_s.at[jj + 1]], gB2, gsem2)
    pltpu.sync_copy(ee_hbm.at[w, j], dbuf)
    pltpu.sync_copy(ee_hbm.at[w, j + 1], dbuf2)
    cp0.wait()
    scale(gB, dbuf)
    pltpu.sync_copy(gB, acc_sh.at[idx_d.at[jj]], add=True)
    cp1.wait()
    scale(gB2, dbuf2)
    pltpu.sync_copy(gB2, acc_sh.at[idx_d.at[jj + 1]], add=True)

  def single(j, jj):
    cp0 = pltpu.async_copy(tabB_hbm.at[idx_s.at[jj]], gB, gsem)
    pltpu.sync_copy(ee_hbm.at[w, j], dbuf)
    cp0.wait()
    scale(gB, dbuf)
    pltpu.sync_copy(gB, acc_sh.at[idx_d.at[jj]], add=True)

  nfull = nchunks // IBLK
  rem = nchunks - nfull * IBLK

  def outer(b, carry):
    pltpu.sync_copy(src_hbm.at[w, pl.ds(b * IBLK, IBLK)], idx_s)
    pltpu.sync_copy(dst_hbm.at[w, pl.ds(b * IBLK, IBLK)], idx_d)

    def inner(pp, c2):
      pair(b * IBLK + pp * 2, pp * 2)
      return c2

    lax.fori_loop(0, IBLK // 2, inner, 0)
    return carry

  lax.fori_loop(0, nfull, outer, 0)
  if rem:
    pltpu.sync_copy(src_hbm.at[w, pl.ds(nfull * IBLK, rem)],
                    idx_s.at[pl.ds(0, rem)])
    pltpu.sync_copy(dst_hbm.at[w, pl.ds(nfull * IBLK, rem)],
                    idx_d.at[pl.ds(0, rem)])

    def inner_rem(pp, c2):
      pair(nfull * IBLK + pp * 2, pp * 2)
      return c2

    lax.fori_loop(0, rem // 2, inner_rem, 0)
    if rem % 2:
      single(nfull * IBLK + rem - 1, rem - 1)
  plsc.subcore_barrier()
  _writeout_shared(acc_sh, acc_out, c)


def _tc_k1(x_ref, w0_ref, degp_ref, hws_ref, dinv_ref):
  deg = degp_ref[0, :NN, 0:1] + degp_ref[1, :NN, 0:1]
  dinv = jnp.where(deg > 0.0, lax.rsqrt(deg), 0.0)
  hw = jnp.dot(x_ref[...], w0_ref[...], preferred_element_type=_f32)
  hws_ref[...] = hw * dinv
  dinv_ref[...] = dinv


def _tc_gcn_post(residual, accp_ref, dinv_ref, b_ref, wn_ref, hprev_ref,
                 h_ref, hwsn_ref):
  acc = accp_ref[0, :NN, :] + accp_ref[1, :NN, :]
  dinv = dinv_ref[...]
  h = jnp.maximum(acc * dinv + b_ref[...], 0.0)
  if residual:
    h = h + hprev_ref[...]
  h_ref[...] = h
  hwsn_ref[...] = jnp.dot(h, wn_ref[...], preferred_element_type=_f32) * dinv


def _tc_k7(accp_ref, dinv_ref, b2_ref, hprev_ref, wg_ref, as_ref, ad_ref,
           tabA_ref, tabB_ref, tabS_ref, tabD_ref, mvec_ref):
  acc = accp_ref[0, :NN, :] + accp_ref[1, :NN, :]
  h3 = hprev_ref[...] + jnp.maximum(acc * dinv_ref[...] + b2_ref[...], 0.0)
  hp = jnp.dot(h3, wg_ref[...], preferred_element_type=_f32)
  al_s = jnp.dot(hp, as_ref[...], preferred_element_type=_f32)
  al_d = jnp.dot(hp, ad_ref[...], preferred_element_type=_f32)
  m = jnp.maximum(jnp.max(al_s) + jnp.max(al_d), 0.0)
  tabA_ref[...] = hp[:, 0:128]
  tabB_ref[...] = hp[:, 128:256]
  tabS_ref[...] = jnp.zeros((NPAD, LANES), _f32)
  tabS_ref[0:NN, 0:4] = al_s
  tabD_ref[...] = jnp.zeros((NPAD, LANES), _f32)
  tabD_ref[0:NN, 0:4] = al_d
  mvec_ref[...] = jnp.full((1, LANES), m, _f32)


def _tc_k10(accA_ref, accB_ref, den_ref, bg_ref, c1_ref, c1b_ref, c2_ref,
            c2b_ref, r1_ref, r1b_ref, r2_ref, r2b_ref,
            cls_ref, rec_ref, h_ref):
  num0 = accA_ref[0, :NN, :] + accA_ref[1, :NN, :]
  num1 = accB_ref[0, :NN, :] + accB_ref[1, :NN, :]
  den = den_ref[0, :NN, 0:4] + den_ref[1, :NN, 0:4]
  h0 = num0[:, 0:HID] / (den[:, 0:1] + 1e-16)
  h1 = num0[:, HID:2 * HID] / (den[:, 1:2] + 1e-16)
  h2 = num1[:, 0:HID] / (den[:, 2:3] + 1e-16)
  h3 = num1[:, HID:2 * HID] / (den[:, 3:4] + 1e-16)
  h = 0.25 * (h0 + h1 + h2 + h3) + bg_ref[...]
  h_ref[...] = h
  hg = jnp.mean(h, axis=0, keepdims=True)
  hc = jnp.maximum(jnp.dot(hg, c1_ref[...], preferred_element_type=_f32)
                   + c1b_ref[...], 0.0)
  cls_ref[...] = jnp.dot(hc, c2_ref[...], preferred_element_type=_f32) + c2b_ref[...]
  hr = jnp.maximum(jnp.dot(h, r1_ref[...], preferred_element_type=_f32)
                   + r1b_ref[...], 0.0)
  rec_ref[...] = jnp.dot(hr, r2_ref[...], preferred_element_type=_f32) + r2b_ref[...]


def kernel(x, edge_index, W0, b0, W1, b1, W2, b2, Wg, a_src, a_dst, bg,
           C1, c1b, C2, c2b, R1, r1b, R2, r2b):
  n_edges = edge_index.shape[1]
  ep_tot = n_edges + NN
  nchunks = -(-ep_tot // (NW * CH))
  ep = NW * CH * nchunks
  npad_e = ep - ep_tot

  loops = jnp.arange(NN, dtype=_i32)
  src = jnp.concatenate([edge_index[0].astype(_i32), loops,
                         jnp.zeros((npad_e,), _i32)])
  dst = jnp.concatenate([edge_index[1].astype(_i32), loops,
                         jnp.full((npad_e,), NN, _i32)])
  srcw = src.reshape(NW, nchunks, CH)
  dstw = dst.reshape(NW, nchunks, CH)

  hmask = (jnp.arange(HEADS * HID)[:, None] // HID
           == jnp.arange(HEADS)[None, :]).astype(_f32)
  As = hmask * a_src.reshape(-1)[:, None]
  Ad = hmask * a_dst.reshape(-1)[:, None]

  deg_call = pl.kernel(
      functools.partial(_deg_body, nchunks),
      out_type=jax.ShapeDtypeStruct((NC, NPAD, LANES), _f32),
      mesh=_mesh,
      scratch_types=[
          pltpu.VMEM((nchunks, CH), _i32),
          pltpu.VMEM((CH, LANES), _f32),
          pltpu.VMEM((CH, LANES), _f32),
          pltpu.VMEM_SHARED((NPAD, LANES), _f32),
      ],
      compiler_params=pltpu.CompilerParams(use_tc_tiling_on_sc=False, needs_layout_passes=False),
      name="sc_degrees",
  )
  degp = deg_call(dstw)

  hws0, dinv = pl.pallas_call(
      _tc_k1,
      out_shape=[jax.ShapeDtypeStruct((NN, HID), _f32),
                 jax.ShapeDtypeStruct((NN, 1), _f32)],
      compiler_params=pltpu.CompilerParams(vmem_limit_bytes=120 * 2**20),
      name="tc_dinv_proj0",
  )(x, W0, degp)

  gcn_call = pl.kernel(
      functools.partial(_gcn_body, nchunks),
      out_type=jax.ShapeDtypeStruct((NC, NPAD, HID), _f32),
      mesh=_mesh,
      scratch_types=[
          pltpu.VMEM((nchunks, CH), _i32),
          pltpu.VMEM((nchunks, CH), _i32),
          pltpu.VMEM((CH, HID), _f32),
          pltpu.VMEM((CH, HID), _f32),
          pltpu.SemaphoreType.DMA,
          pltpu.SemaphoreType.DMA,
          pltpu.VMEM_SHARED((NPAD, HID), _f32),
      ],
      compiler_params=pltpu.CompilerParams(use_tc_tiling_on_sc=False, needs_layout_passes=False),
      name="sc_gcn_edges",
  )

  def gcn_post(residual, accp, b, wn, hprev):
    return pl.pallas_call(
        functools.partial(_tc_gcn_post, residual),
        out_shape=[jax.ShapeDtypeStruct((NN, HID), _f32),
                   jax.ShapeDtypeStruct((NN, HID), _f32)],
        compiler_params=pltpu.CompilerParams(vmem_limit_bytes=120 * 2**20),
      name="tc_gcn_post",
    )(accp, dinv, b.reshape(1, HID), wn, hprev)

  accp0 = gcn_call(srcw, dstw, hws0)
  h1, hws1 = gcn_post(False, accp0, b0, W1, hws0)
  accp1 = gcn_call(srcw, dstw, hws1)
  h2, hws2 = gcn_post(True, accp1, b1, W2, h1)
  accp2 = gcn_call(srcw, dstw, hws2)

  tabA, tabB, tabS, tabD, mvec = pl.pallas_call(
      _tc_k7,
      out_shape=[jax.ShapeDtypeStruct((NN, 2 * HID), _f32),
                 jax.ShapeDtypeStruct((NN, 2 * HID), _f32),
                 jax.ShapeDtypeStruct((NPAD, LANES), _f32),
                 jax.ShapeDtypeStruct((NPAD, LANES), _f32),
                 jax.ShapeDtypeStruct((1, LANES), _f32)],
      compiler_params=pltpu.CompilerParams(vmem_limit_bytes=120 * 2**20),
      name="tc_gat_prep",
  )(accp2, dinv, b2.reshape(1, HID), h2, Wg, As, Ad)

  gata_call = pl.kernel(
      functools.partial(_gata_body, nchunks),
      out_type=[jax.ShapeDtypeStruct((NC, NPAD, 2 * HID), _f32),
                jax.ShapeDtypeStruct((NC, NPAD, LANES), _f32),
                jax.ShapeDtypeStruct((NW, nchunks, CH, LANES), _f32)],
      mesh=_mesh,
      scratch_types=[
          pltpu.VMEM((IBLK, CH), _i32),
          pltpu.VMEM((IBLK, CH), _i32),
          pltpu.VMEM((CH, 2 * HID), _f32),
          pltpu.VMEM((CH, LANES), _f32),
          pltpu.VMEM((CH, LANES), _f32),
          pltpu.VMEM((CH, LANES), _f32),
          pltpu.VMEM((1, LANES), _f32),
          pltpu.SemaphoreType.DMA,
          pltpu.SemaphoreType.DMA,
          pltpu.SemaphoreType.DMA,
          pltpu.VMEM_SHARED((NPAD, 2 * HID), _f32),
          pltpu.VMEM_SHARED((NPAD, LANES), _f32),
      ],
      compiler_params=pltpu.CompilerParams(use_tc_tiling_on_sc=False, needs_layout_passes=False),
      name="sc_gat_a",
  )
  accA, denp, ee = gata_call(srcw, dstw, tabA, tabS, tabD, mvec)

  gatb_call = pl.kernel(
      functools.partial(_gatb_body, nchunks),
      out_type=jax.ShapeDtypeStruct((NC, NPAD, 2 * HID), _f32),
      mesh=_mesh,
      scratch_types=[
          pltpu.VMEM((IBLK, CH), _i32),
          pltpu.VMEM((IBLK, CH), _i32),
          pltpu.VMEM((CH, 2 * HID), _f32),
          pltpu.VMEM((CH, 2 * HID), _f32),
          pltpu.VMEM((CH, LANES), _f32),
          pltpu.VMEM((CH, LANES), _f32),
          pltpu.SemaphoreType.DMA,
          pltpu.SemaphoreType.DMA,
          pltpu.VMEM_SHARED((NPAD, 2 * HID), _f32),
      ],
      compiler_params=pltpu.CompilerParams(use_tc_tiling_on_sc=False, needs_layout_passes=False),
      name="sc_gat_b",
  )
  accB = gatb_call(srcw, dstw, tabB, ee)

  cls, rec, h = pl.pallas_call(
      _tc_k10,
      out_shape=[jax.ShapeDtypeStruct((1, 2), _f32),
                 jax.ShapeDtypeStruct((NN, D_IN), _f32),
                 jax.ShapeDtypeStruct((NN, HID), _f32)],
      compiler_params=pltpu.CompilerParams(vmem_limit_bytes=120 * 2**20),
      name="tc_final",
  )(accA, accB, denp, bg.reshape(1, HID), C1, c1b.reshape(1, HID // 2),
    C2, c2b.reshape(1, 2), R1, r1b.reshape(1, HID), R2, r2b.reshape(1, D_IN))
  return (cls, rec, h)


# speedup vs baseline: 39.9571x; 1.0250x over previous
import functools

import jax
import jax.numpy as jnp
from jax import lax
from jax.experimental import pallas as pl
from jax.experimental.pallas import tpu as pltpu
from jax.experimental.pallas import tpu_sc as plsc

NN = 10000
D_IN = 128
HID = 64
HEADS = 4

NC, NS, LANES = 2, 16, 16
NW = NC * NS
CH = 128
NPAD = 10112
IBLK = 32
GDEPTH = 4
RPT = NPAD // NS

_mesh = plsc.VectorSubcoreMesh(
    core_axis_name="c", subcore_axis_name="s", num_cores=NC, num_subcores=NS)

_f32 = jnp.float32
_i32 = jnp.int32


def _worker_ids():
  c = lax.axis_index("c")
  s = lax.axis_index("s")
  return c, s, c * NS + s


def _zero_vmem(buf, rows, cols):
  z = jnp.zeros((LANES,), _f32)

  def body(i, carry):
    for q in range(cols // LANES):
      buf[i, pl.ds(q * LANES, LANES)] = z
    return carry

  lax.fori_loop(0, rows, body, 0)


def _zero_shared_slice(sh, zbuf):
  s = lax.axis_index("s")
  base = s * RPT
  n_full = RPT // CH
  rem = RPT - n_full * CH
  for m in range(n_full):
    pltpu.sync_copy(zbuf, sh.at[pl.ds(base + m * CH, CH)])
  if rem:
    pltpu.sync_copy(zbuf.at[pl.ds(0, rem)], sh.at[pl.ds(base + n_full * CH, rem)])


def _writeout_shared(sh, out_hbm, c):
  s = lax.axis_index("s")
  base = s * RPT
  pltpu.sync_copy(sh.at[pl.ds(base, RPT)], out_hbm.at[c, pl.ds(base, RPT)])


def _deg_body(nchunks, dst_hbm, out_hbm, idx_d, ones_v, zbuf, deg_sh):
  c, s, w = _worker_ids()
  one = jnp.ones((LANES,), _f32)

  def initbody(i, carry):
    ones_v[i, pl.ds(0, LANES)] = one
    zbuf[i, pl.ds(0, LANES)] = one * 0.0
    return carry

  lax.fori_loop(0, CH, initbody, 0)
  pltpu.sync_copy(dst_hbm.at[w], idx_d)
  _zero_shared_slice(deg_sh, zbuf)
  plsc.subcore_barrier()

  def body(j, carry):
    pltpu.sync_copy(ones_v, deg_sh.at[idx_d.at[j]], add=True)
    return carry

  lax.fori_loop(0, nchunks, body, 0)
  plsc.subcore_barrier()
  _writeout_shared(deg_sh, out_hbm, c)


def _gcn_body(nchunks, src_hbm, dst_hbm, tab_hbm, out_hbm,
              idx_s, idx_d, g0, g1, g2, g3, s0, s1, s2, s3, acc_sh):
  gbufs = [g0, g1, g2, g3]
  gsems = [s0, s1, s2, s3]
  c, s, w = _worker_ids()
  _zero_vmem(gbufs[0], CH, HID)
  pltpu.sync_copy(src_hbm.at[w], idx_s)
  pltpu.sync_copy(dst_hbm.at[w], idx_d)
  _zero_shared_slice(acc_sh, gbufs[0])
  plsc.subcore_barrier()

  ngroups = nchunks // GDEPTH

  def body(p, carry):
    j = p * GDEPTH
    cps = [pltpu.async_copy(tab_hbm.at[idx_s.at[j + q]], gbufs[q], gsems[q])
           for q in range(GDEPTH)]
    for q in range(GDEPTH):
      cps[q].wait()
      pltpu.sync_copy(gbufs[q], acc_sh.at[idx_d.at[j + q]], add=True)
    return carry

  lax.fori_loop(0, ngroups, body, 0)
  for j in range(ngroups * GDEPTH, nchunks):
    pltpu.async_copy(tab_hbm.at[idx_s.at[j]], gbufs[0], gsems[0]).wait()
    pltpu.sync_copy(gbufs[0], acc_sh.at[idx_d.at[j]], add=True)
  plsc.subcore_barrier()
  _writeout_shared(acc_sh, out_hbm, c)


def _gata_body(nchunks, src_hbm, dst_hbm, tabA_hbm, tabS_hbm, tabD_hbm, mvec_hbm,
               acc_out, den_out, ee_out,
               idx_s, idx_d, gA, gS, gD, dbuf, mv, gsem, ssem, dsem,
               acc_sh, den_sh):
  c, s, w = _worker_ids()
  pltpu.sync_copy(mvec_hbm, mv)
  _zero_vmem(gA, CH, 2 * HID)
  _zero_shared_slice(acc_sh, gA)
  _zero_vmem(dbuf, CH, LANES)
  _zero_shared_slice(den_sh, dbuf)
  plsc.subcore_barrier()
  mvv = mv[0, pl.ds(0, LANES)]

  def chunk(j, jj):
    cpA = pltpu.async_copy(tabA_hbm.at[idx_s.at[jj]], gA, gsem)
    cpS = pltpu.async_copy(tabS_hbm.at[idx_s.at[jj]], gS, ssem)
    cpD = pltpu.async_copy(tabD_hbm.at[idx_d.at[jj]], gD, dsem)
    cpS.wait()
    cpD.wait()
    for k in range(CH // LANES):
      rows = lax.iota(_i32, LANES) + (LANES * k)
      for h in range(HEADS):
        colS = jnp.full((LANES,), h, _i32)
        colD = jnp.full((LANES,), h, _i32)
        sv = plsc.load_gather(gS, [rows, colS])
        dv = plsc.load_gather(gD, [rows, colD])
        es = sv + dv
        e = jnp.where(es > 0.0, es, es * 0.2)
        plsc.store_scatter(dbuf, [rows, colD], jnp.exp(e - mvv))
    cpA.wait()
    hsel = jnp.bitwise_and(lax.iota(_i32, LANES), 3)

    def sc_body(i, carry2):
      ev = plsc.load_gather(dbuf, [jnp.full((LANES,), i, _i32), hsel])
      for h in range(2):
        svec = jnp.full((LANES,), ev[h], _f32)
        for q in range(HID // LANES):
          slc = pl.ds(h * HID + q * LANES, LANES)
          gA[i, slc] = gA[i, slc] * svec
      return carry2

    lax.fori_loop(0, CH, sc_body, 0)
    pltpu.sync_copy(gA, acc_sh.at[idx_d.at[jj]], add=True)
    pltpu.sync_copy(dbuf, den_sh.at[idx_d.at[jj]], add=True)
    pltpu.sync_copy(dbuf, ee_out.at[w, j])

  nfull = nchunks // IBLK
  rem = nchunks - nfull * IBLK

  def outer(b, carry):
    pltpu.sync_copy(src_hbm.at[w, pl.ds(b * IBLK, IBLK)], idx_s)
    pltpu.sync_copy(dst_hbm.at[w, pl.ds(b * IBLK, IBLK)], idx_d)

    def inner(jj, c2):
      chunk(b * IBLK + jj, jj)
      return c2

    lax.fori_loop(0, IBLK, inner, 0)
    return carry

  lax.fori_loop(0, nfull, outer, 0)
  if rem:
    pltpu.sync_copy(src_hbm.at[w, pl.ds(nfull * IBLK, rem)],
                    idx_s.at[pl.ds(0, rem)])
    pltpu.sync_copy(dst_hbm.at[w, pl.ds(nfull * IBLK, rem)],
                    idx_d.at[pl.ds(0, rem)])

    def inner_rem(jj, c2):
      chunk(nfull * IBLK + jj, jj)
      return c2

    lax.fori_loop(0, rem, inner_rem, 0)
  plsc.subcore_barrier()
  _writeout_shared(acc_sh, acc_out, c)
  _writeout_shared(den_sh, den_out, c)


def _gatb_body(nchunks, src_hbm, dst_hbm, tabB_hbm, ee_hbm, acc_out,
               idx_s, idx_d, gB, gB2, dbuf, dbuf2, gsem, gsem2, acc_sh):
  c, s, w = _worker_ids()
  _zero_vmem(gB, CH, 2 * HID)
  _zero_shared_slice(acc_sh, gB)
  plsc.subcore_barrier()
  hsel = jnp.bitwise_and(lax.iota(_i32, LANES), 3)

  def scale(gX, dX):
    def sc_body(i, carry2):
      ev = plsc.load_gather(dX, [jnp.full((LANES,), i, _i32), hsel])
      for h in range(2):
        svec = jnp.full((LANES,), ev[2 + h], _f32)
        for q in range(HID // LANES):
          slc = pl.ds(h * HID + q * LANES, LANES)
          gX[i, slc] = gX[i, slc] * svec
      return carry2

    lax.fori_loop(0, CH, sc_body, 0)

  def pair(j, jj):
    cp0 = pltpu.async_copy(tabB_hbm.at[idx_s.at[jj]], gB, gsem)
    cp1 = pltpu.async_copy(tabB_hbm.at[idx_s.at[jj + 1]], gB2, gsem2)
    pltpu.sync_copy(ee_hbm.at[w, j], dbuf)
    pltpu.sync_copy(ee_hbm.at[w, j + 1], dbuf2)
    cp0.wait()
    scale(gB, dbuf)
    pltpu.sync_copy(gB, acc_sh.at[idx_d.at[jj]], add=True)
    cp1.wait()
    scale(gB2, dbuf2)
    pltpu.sync_copy(gB2, acc_sh.at[idx_d.at[jj + 1]], add=True)

  def single(j, jj):
    cp0 = pltpu.async_copy(tabB_hbm.at[idx_s.at[jj]], gB, gsem)
    pltpu.sync_copy(ee_hbm.at[w, j], dbuf)
    cp0.wait()
    scale(gB, dbuf)
    pltpu.sync_copy(gB, acc_sh.at[idx_d.at[jj]], add=True)

  nfull = nchunks // IBLK
  rem = nchunks - nfull * IBLK

  def outer(b, carry):
    pltpu.sync_copy(src_hbm.at[w, pl.ds(b * IBLK, IBLK)], idx_s)
    pltpu.sync_copy(dst_hbm.at[w, pl.ds(b * IBLK, IBLK)], idx_d)

    def inner(pp, c2):
      pair(b * IBLK + pp * 2, pp * 2)
      return c2

    lax.fori_loop(0, IBLK // 2, inner, 0)
    return carry

  lax.fori_loop(0, nfull, outer, 0)
  if rem:
    pltpu.sync_copy(src_hbm.at[w, pl.ds(nfull * IBLK, rem)],
                    idx_s.at[pl.ds(0, rem)])
    pltpu.sync_copy(dst_hbm.at[w, pl.ds(nfull * IBLK, rem)],
                    idx_d.at[pl.ds(0, rem)])

    def inner_rem(pp, c2):
      pair(nfull * IBLK + pp * 2, pp * 2)
      return c2

    lax.fori_loop(0, rem // 2, inner_rem, 0)
    if rem % 2:
      single(nfull * IBLK + rem - 1, rem - 1)
  plsc.subcore_barrier()
  _writeout_shared(acc_sh, acc_out, c)


def _tc_k1(x_ref, w0_ref, degp_ref, hws_ref, dinv_ref):
  deg = degp_ref[0, :NN, 0:1] + degp_ref[1, :NN, 0:1]
  dinv = jnp.where(deg > 0.0, lax.rsqrt(deg), 0.0)
  hw = jnp.dot(x_ref[...], w0_ref[...], preferred_element_type=_f32)
  hws_ref[...] = hw * dinv
  dinv_ref[...] = dinv


def _tc_gcn_post(residual, accp_ref, dinv_ref, b_ref, wn_ref, hprev_ref,
                 h_ref, hwsn_ref):
  acc = accp_ref[0, :NN, :] + accp_ref[1, :NN, :]
  dinv = dinv_ref[...]
  h = jnp.maximum(acc * dinv + b_ref[...], 0.0)
  if residual:
    h = h + hprev_ref[...]
  h_ref[...] = h
  hwsn_ref[...] = jnp.dot(h, wn_ref[...], preferred_element_type=_f32) * dinv


def _tc_k7(accp_ref, dinv_ref, b2_ref, hprev_ref, wg_ref, as_ref, ad_ref,
           tabA_ref, tabB_ref, tabS_ref, tabD_ref, mvec_ref):
  acc = accp_ref[0, :NN, :] + accp_ref[1, :NN, :]
  h3 = hprev_ref[...] + jnp.maximum(acc * dinv_ref[...] + b2_ref[...], 0.0)
  hp = jnp.dot(h3, wg_ref[...], preferred_element_type=_f32)
  al_s = jnp.dot(hp, as_ref[...], preferred_element_type=_f32)
  al_d = jnp.dot(hp, ad_ref[...], preferred_element_type=_f32)
  m = jnp.maximum(jnp.max(al_s) + jnp.max(al_d), 0.0)
  tabA_ref[...] = hp[:, 0:128]
  tabB_ref[...] = hp[:, 128:256]
  tabS_ref[...] = jnp.zeros((NPAD, LANES), _f32)
  tabS_ref[0:NN, 0:4] = al_s
  tabD_ref[...] = jnp.zeros((NPAD, LANES), _f32)
  tabD_ref[0:NN, 0:4] = al_d
  mvec_ref[...] = jnp.full((1, LANES), m, _f32)


def _tc_k10(accA_ref, accB_ref, den_ref, bg_ref, c1_ref, c1b_ref, c2_ref,
            c2b_ref, r1_ref, r1b_ref, r2_ref, r2b_ref,
            cls_ref, rec_ref, h_ref):
  num0 = accA_ref[0, :NN, :] + accA_ref[1, :NN, :]
  num1 = accB_ref[0, :NN, :] + accB_ref[1, :NN, :]
  den = den_ref[0, :NN, 0:4] + den_ref[1, :NN, 0:4]
  h0 = num0[:, 0:HID] / (den[:, 0:1] + 1e-16)
  h1 = num0[:, HID:2 * HID] / (den[:, 1:2] + 1e-16)
  h2 = num1[:, 0:HID] / (den[:, 2:3] + 1e-16)
  h3 = num1[:, HID:2 * HID] / (den[:, 3:4] + 1e-16)
  h = 0.25 * (h0 + h1 + h2 + h3) + bg_ref[...]
  h_ref[...] = h
  hg = jnp.mean(h, axis=0, keepdims=True)
  hc = jnp.maximum(jnp.dot(hg, c1_ref[...], preferred_element_type=_f32)
                   + c1b_ref[...], 0.0)
  cls_ref[...] = jnp.dot(hc, c2_ref[...], preferred_element_type=_f32) + c2b_ref[...]
  hr = jnp.maximum(jnp.dot(h, r1_ref[...], preferred_element_type=_f32)
                   + r1b_ref[...], 0.0)
  rec_ref[...] = jnp.dot(hr, r2_ref[...], preferred_element_type=_f32) + r2b_ref[...]


def kernel(x, edge_index, W0, b0, W1, b1, W2, b2, Wg, a_src, a_dst, bg,
           C1, c1b, C2, c2b, R1, r1b, R2, r2b):
  n_edges = edge_index.shape[1]
  ep_tot = n_edges + NN
  nchunks = -(-ep_tot // (NW * CH))
  ep = NW * CH * nchunks
  npad_e = ep - ep_tot

  loops = jnp.arange(NN, dtype=_i32)
  src = jnp.concatenate([edge_index[0].astype(_i32), loops,
                         jnp.zeros((npad_e,), _i32)])
  dst = jnp.concatenate([edge_index[1].astype(_i32), loops,
                         jnp.full((npad_e,), NN, _i32)])
  srcw = src.reshape(NW, nchunks, CH)
  dstw = dst.reshape(NW, nchunks, CH)

  hmask = (jnp.arange(HEADS * HID)[:, None] // HID
           == jnp.arange(HEADS)[None, :]).astype(_f32)
  As = hmask * a_src.reshape(-1)[:, None]
  Ad = hmask * a_dst.reshape(-1)[:, None]

  deg_call = pl.kernel(
      functools.partial(_deg_body, nchunks),
      out_type=jax.ShapeDtypeStruct((NC, NPAD, LANES), _f32),
      mesh=_mesh,
      scratch_types=[
          pltpu.VMEM((nchunks, CH), _i32),
          pltpu.VMEM((CH, LANES), _f32),
          pltpu.VMEM((CH, LANES), _f32),
          pltpu.VMEM_SHARED((NPAD, LANES), _f32),
      ],
      compiler_params=pltpu.CompilerParams(use_tc_tiling_on_sc=False, needs_layout_passes=False),
      name="sc_degrees",
  )
  degp = deg_call(dstw)

  hws0, dinv = pl.pallas_call(
      _tc_k1,
      out_shape=[jax.ShapeDtypeStruct((NN, HID), _f32),
                 jax.ShapeDtypeStruct((NN, 1), _f32)],
      compiler_params=pltpu.CompilerParams(vmem_limit_bytes=120 * 2**20),
      name="tc_dinv_proj0",
  )(x, W0, degp)

  gcn_call = pl.kernel(
      functools.partial(_gcn_body, nchunks),
      out_type=jax.ShapeDtypeStruct((NC, NPAD, HID), _f32),
      mesh=_mesh,
      scratch_types=[
          pltpu.VMEM((nchunks, CH), _i32),
          pltpu.VMEM((nchunks, CH), _i32),
          pltpu.VMEM((CH, HID), _f32),
          pltpu.VMEM((CH, HID), _f32),
          pltpu.VMEM((CH, HID), _f32),
          pltpu.VMEM((CH, HID), _f32),
          pltpu.SemaphoreType.DMA,
          pltpu.SemaphoreType.DMA,
          pltpu.SemaphoreType.DMA,
          pltpu.SemaphoreType.DMA,
          pltpu.VMEM_SHARED((NPAD, HID), _f32),
      ],
      compiler_params=pltpu.CompilerParams(use_tc_tiling_on_sc=False, needs_layout_passes=False),
      name="sc_gcn_edges",
  )

  def gcn_post(residual, accp, b, wn, hprev):
    return pl.pallas_call(
        functools.partial(_tc_gcn_post, residual),
        out_shape=[jax.ShapeDtypeStruct((NN, HID), _f32),
                   jax.ShapeDtypeStruct((NN, HID), _f32)],
        compiler_params=pltpu.CompilerParams(vmem_limit_bytes=120 * 2**20),
      name="tc_gcn_post",
    )(accp, dinv, b.reshape(1, HID), wn, hprev)

  accp0 = gcn_call(srcw, dstw, hws0)
  h1, hws1 = gcn_post(False, accp0, b0, W1, hws0)
  accp1 = gcn_call(srcw, dstw, hws1)
  h2, hws2 = gcn_post(True, accp1, b1, W2, h1)
  accp2 = gcn_call(srcw, dstw, hws2)

  tabA, tabB, tabS, tabD, mvec = pl.pallas_call(
      _tc_k7,
      out_shape=[jax.ShapeDtypeStruct((NN, 2 * HID), _f32),
                 jax.ShapeDtypeStruct((NN, 2 * HID), _f32),
                 jax.ShapeDtypeStruct((NPAD, LANES), _f32),
                 jax.ShapeDtypeStruct((NPAD, LANES), _f32),
                 jax.ShapeDtypeStruct((1, LANES), _f32)],
      compiler_params=pltpu.CompilerParams(vmem_limit_bytes=120 * 2**20),
      name="tc_gat_prep",
  )(accp2, dinv, b2.reshape(1, HID), h2, Wg, As, Ad)

  gata_call = pl.kernel(
      functools.partial(_gata_body, nchunks),
      out_type=[jax.ShapeDtypeStruct((NC, NPAD, 2 * HID), _f32),
                jax.ShapeDtypeStruct((NC, NPAD, LANES), _f32),
                jax.ShapeDtypeStruct((NW, nchunks, CH, LANES), _f32)],
      mesh=_mesh,
      scratch_types=[
          pltpu.VMEM((IBLK, CH), _i32),
          pltpu.VMEM((IBLK, CH), _i32),
          pltpu.VMEM((CH, 2 * HID), _f32),
          pltpu.VMEM((CH, LANES), _f32),
          pltpu.VMEM((CH, LANES), _f32),
          pltpu.VMEM((CH, LANES), _f32),
          pltpu.VMEM((1, LANES), _f32),
          pltpu.SemaphoreType.DMA,
          pltpu.SemaphoreType.DMA,
          pltpu.SemaphoreType.DMA,
          pltpu.VMEM_SHARED((NPAD, 2 * HID), _f32),
          pltpu.VMEM_SHARED((NPAD, LANES), _f32),
      ],
      compiler_params=pltpu.CompilerParams(use_tc_tiling_on_sc=False, needs_layout_passes=False),
      name="sc_gat_a",
  )
  accA, denp, ee = gata_call(srcw, dstw, tabA, tabS, tabD, mvec)

  gatb_call = pl.kernel(
      functools.partial(_gatb_body, nchunks),
      out_type=jax.ShapeDtypeStruct((NC, NPAD, 2 * HID), _f32),
      mesh=_mesh,
      scratch_types=[
          pltpu.VMEM((IBLK, CH), _i32),
          pltpu.VMEM((IBLK, CH), _i32),
          pltpu.VMEM((CH, 2 * HID), _f32),
          pltpu.VMEM((CH, 2 * HID), _f32),
          pltpu.VMEM((CH, LANES), _f32),
          pltpu.VMEM((CH, LANES), _f32),
          pltpu.SemaphoreType.DMA,
          pltpu.SemaphoreType.DMA,
          pltpu.VMEM_SHARED((NPAD, 2 * HID), _f32),
      ],
      compiler_params=pltpu.CompilerParams(use_tc_tiling_on_sc=False, needs_layout_passes=False),
      name="sc_gat_b",
  )
  accB = gatb_call(srcw, dstw, tabB, ee)

  cls, rec, h = pl.pallas_call(
      _tc_k10,
      out_shape=[jax.ShapeDtypeStruct((1, 2), _f32),
                 jax.ShapeDtypeStruct((NN, D_IN), _f32),
                 jax.ShapeDtypeStruct((NN, HID), _f32)],
      compiler_params=pltpu.CompilerParams(vmem_limit_bytes=120 * 2**20),
      name="tc_final",
  )(accA, accB, denp, bg.reshape(1, HID), C1, c1b.reshape(1, HID // 2),
    C2, c2b.reshape(1, 2), R1, r1b.reshape(1, HID), R2, r2b.reshape(1, D_IN))
  return (cls, rec, h)
